# baseline (speedup 1.0000x reference)
import functools

import jax
import jax.numpy as jnp
from jax import lax
from jax.experimental import pallas as pl
from jax.experimental.pallas import tpu as pltpu
from jax.experimental.pallas import tpu_sc as plsc

_NC = 2
_NS = 16
_NW = _NC * _NS

_B = 16384
_D = 64
_ROWS_PER_TILE = _D // _NW
_V = 100001
_OUT_CHUNK = 2048


def _embed_body(idx_hbm, tab_hbm, out_hbm, idx_v, row_v, stage_v, sem, isem):
    wid = lax.axis_index("s") * _NC + lax.axis_index("c")
    idx_cp = pltpu.async_copy(idx_hbm, idx_v, isem)
    zeros16 = jnp.zeros((16,), jnp.int32)
    pending = [None, None]
    for p in range(_ROWS_PER_TILE):
        d = wid + _NW * p
        pltpu.sync_copy(tab_hbm.at[pl.ds(d, 1)], row_v)
        if p == 0:
            idx_cp.wait()
        for h in range(_B // _OUT_CHUNK):
            b = (p * (_B // _OUT_CHUNK) + h) % 2
            if pending[b] is not None:
                pending[b].wait()

            @plsc.parallel_loop(0, _OUT_CHUNK // 16, unroll=16)
            def step(i, _h=h, _b=b):
                idx16 = idx_v[pl.ds(_h * _OUT_CHUNK + i * 16, 16)]
                vals = plsc.load_gather(row_v, [zeros16, idx16])
                stage_v[pl.ds(_b * _OUT_CHUNK + i * 16, 16)] = vals

            pending[b] = pltpu.async_copy(
                stage_v.at[pl.ds(b * _OUT_CHUNK, _OUT_CHUNK)],
                out_hbm.at[d, pl.ds(h * _OUT_CHUNK, _OUT_CHUNK)],
                sem,
            )
    for c in pending:
        if c is not None:
            c.wait()


@jax.jit
def _embed(idx, tab_t):
    run = pl.kernel(
        _embed_body,
        out_type=jax.ShapeDtypeStruct((_D, _B), jnp.float32),
        mesh=plsc.VectorSubcoreMesh(
            core_axis_name="c", subcore_axis_name="s",
            num_cores=_NC, num_subcores=_NS,
        ),
        scratch_types=[
            pltpu.VMEM((_B,), jnp.int32),
            pltpu.VMEM((1, _V), jnp.float32),
            pltpu.VMEM((2 * _OUT_CHUNK,), jnp.float32),
            pltpu.SemaphoreType.DMA,
            pltpu.SemaphoreType.DMA,
        ],
        compiler_params=pltpu.CompilerParams(needs_layout_passes=False, disable_bounds_checks=True, disable_semaphore_checks=True),
    )
    return run(idx, tab_t)


def kernel(x, table):
    out_t = _embed(x.astype(jnp.int32), table.T)
    return out_t.T


# speedup vs baseline: 1.0482x; 1.0482x over previous
import functools

import jax
import jax.numpy as jnp
from jax import lax
from jax.experimental import pallas as pl
from jax.experimental.pallas import tpu as pltpu
from jax.experimental.pallas import tpu_sc as plsc

_NC = 2
_NS = 16
_NW = _NC * _NS

_B = 16384
_D = 64
_ROWS_PER_TILE = _D // _NW
_V = 100001
_OUT_CHUNK = 4096


def _embed_body(idx_hbm, tab_hbm, out_hbm, idx_v, row_v, stage_v, sem, isem):
    wid = lax.axis_index("s") * _NC + lax.axis_index("c")
    idx_cp = pltpu.async_copy(idx_hbm, idx_v, isem)
    zeros16 = jnp.zeros((16,), jnp.int32)
    pending = [None, None]
    for p in range(_ROWS_PER_TILE):
        d = wid + _NW * p
        pltpu.sync_copy(tab_hbm.at[pl.ds(d, 1)], row_v)
        if p == 0:
            idx_cp.wait()
        for h in range(_B // _OUT_CHUNK):
            b = (p * (_B // _OUT_CHUNK) + h) % 2
            if pending[b] is not None:
                pending[b].wait()

            @plsc.parallel_loop(0, _OUT_CHUNK // 16, unroll=8)
            def step(i, _h=h, _b=b):
                idx16 = idx_v[pl.ds(_h * _OUT_CHUNK + i * 16, 16)]
                vals = plsc.load_gather(row_v, [zeros16, idx16])
                stage_v[pl.ds(_b * _OUT_CHUNK + i * 16, 16)] = vals

            pending[b] = pltpu.async_copy(
                stage_v.at[pl.ds(b * _OUT_CHUNK, _OUT_CHUNK)],
                out_hbm.at[d, pl.ds(h * _OUT_CHUNK, _OUT_CHUNK)],
                sem,
            )
    for c in pending:
        if c is not None:
            c.wait()


@jax.jit
def _embed(idx, tab_t):
    run = pl.kernel(
        _embed_body,
        out_type=jax.ShapeDtypeStruct((_D, _B), jnp.float32),
        mesh=plsc.VectorSubcoreMesh(
            core_axis_name="c", subcore_axis_name="s",
            num_cores=_NC, num_subcores=_NS,
        ),
        scratch_types=[
            pltpu.VMEM((_B,), jnp.int32),
            pltpu.VMEM((1, _V), jnp.float32),
            pltpu.VMEM((2 * _OUT_CHUNK,), jnp.float32),
            pltpu.SemaphoreType.DMA,
            pltpu.SemaphoreType.DMA,
        ],
        compiler_params=pltpu.CompilerParams(needs_layout_passes=False, disable_bounds_checks=True, disable_semaphore_checks=True),
    )
    return run(idx, tab_t)


def kernel(x, table):
    out_t = _embed(x.astype(jnp.int32), table.T)
    return out_t.T


# speedup vs baseline: 1.0514x; 1.0030x over previous
import functools

import jax
import jax.numpy as jnp
from jax import lax
from jax.experimental import pallas as pl
from jax.experimental.pallas import tpu as pltpu
from jax.experimental.pallas import tpu_sc as plsc

_NC = 2
_NS = 16
_NW = _NC * _NS

_B = 16384
_D = 64
_ROWS_PER_TILE = _D // _NW
_V = 100001
_OUT_CHUNK = 4096


def _embed_body(idx_hbm, tab_hbm, out_hbm, idx_v, row_v, stage_v, sem, isem):
    wid = lax.axis_index("s") * _NC + lax.axis_index("c")
    idx_cp = pltpu.async_copy(idx_hbm, idx_v, isem)
    zeros16 = jnp.zeros((16,), jnp.int32)
    pending = [None, None]
    for p in range(_ROWS_PER_TILE):
        d = wid + _NW * p
        pltpu.sync_copy(tab_hbm.at[pl.ds(d, 1)], row_v)
        if p == 0:
            idx_cp.wait()
        for h in range(_B // _OUT_CHUNK):
            b = (p * (_B // _OUT_CHUNK) + h) % 2
            if pending[b] is not None:
                pending[b].wait()

            @plsc.parallel_loop(0, _OUT_CHUNK // 16, unroll=8)
            def step(i, _h=h, _b=b):
                idx16 = idx_v[pl.ds(_h * _OUT_CHUNK + i * 16, 16)]
                vals = plsc.load_gather(row_v, [zeros16, idx16])
                stage_v[pl.ds(_b * _OUT_CHUNK + i * 16, 16)] = vals

            pending[b] = pltpu.async_copy(
                stage_v.at[pl.ds(b * _OUT_CHUNK, _OUT_CHUNK)],
                out_hbm.at[d, pl.ds(h * _OUT_CHUNK, _OUT_CHUNK)],
                sem,
            )
    for c in pending:
        if c is not None:
            c.wait()


@jax.jit
def _embed(idx, tab_t):
    run = pl.kernel(
        _embed_body,
        out_type=jax.ShapeDtypeStruct((_D, _B), jnp.float32),
        mesh=plsc.VectorSubcoreMesh(
            core_axis_name="c", subcore_axis_name="s",
            num_cores=_NC, num_subcores=_NS,
        ),
        scratch_types=[
            pltpu.VMEM((_B,), jnp.int32),
            pltpu.VMEM((1, _V), jnp.float32),
            pltpu.VMEM((2 * _OUT_CHUNK,), jnp.float32),
            pltpu.SemaphoreType.DMA,
            pltpu.SemaphoreType.DMA,
        ],
        compiler_params=pltpu.CompilerParams(needs_layout_passes=False, disable_bounds_checks=True, disable_semaphore_checks=True, skip_device_barrier=True),
    )
    return run(idx, tab_t)


def kernel(x, table):
    out_t = _embed(x.astype(jnp.int32), table.T)
    return out_t.T
